# baseline (speedup 1.0000x reference)
import jax, jax.numpy as jnp
from jax.experimental import pallas as pl

def _copy_body(x_ref, o_ref):
    o_ref[...] = x_ref[...]

def kernel(x, edge_attr, visited, params, edge_index, solution, selected):
    p = params
    def _lin(h, W, b): return h @ W.T + b
    def _bn(h, g, b):
        m = h.mean(0); v = h.var(0)
        return g * (h - m) / jnp.sqrt(v + 1e-5) + b
    def _silu(t): return t * jax.nn.sigmoid(t)
    def _segmean(src, idx, n):
        s = jax.ops.segment_sum(src, idx, num_segments=n)
        c = jax.ops.segment_sum(jnp.ones((src.shape[0],), src.dtype), idx, num_segments=n)
        return s / jnp.maximum(c, 1.0)[:, None]
    ei0 = edge_index[0]; ei1 = edge_index[1]; n = x.shape[0]
    x0 = pl.pallas_call(_copy_body, out_shape=jax.ShapeDtypeStruct(x.shape, x.dtype))(x)
    h = _silu(_lin(x0, p["Wv0"], p["bv0"]))
    w = _silu(_lin(edge_attr, p["We0"], p["be0"]))
    for i in range(12):
        xx0 = h; w0 = w
        x1 = _lin(xx0, p["Wv1"][i], p["bv1"][i]); x2 = _lin(xx0, p["Wv2"][i], p["bv2"][i])
        x3 = _lin(xx0, p["Wv3"][i], p["bv3"][i]); x4 = _lin(xx0, p["Wv4"][i], p["bv4"][i])
        w1 = _lin(w0, p["We"][i], p["be"][i]); w2 = jax.nn.sigmoid(w0)
        agg = _segmean(w2 * x2[ei1], ei0, n)
        h = xx0 + _silu(_bn(x1 + agg, p["vbn_g"][i], p["vbn_b"][i]))
        w = w0 + _silu(_bn(w1 + x3[ei0] + x4[ei1], p["ebn_g"][i], p["ebn_b"][i]))
    x_emb = h; e_emb = w
    sol = jnp.concatenate([solution[:, 0:1], selected], axis=-1)
    bsz = sol.shape[0]
    x_depot = jnp.broadcast_to(x_emb[0][None, None, :], (bsz, 1, x_emb.shape[1]))
    x_first = x_emb[sol[:, 0]][:, None, :]; x_last = x_emb[sol[:, 1]][:, None, :]
    x_vis = x_emb; xm = e_emb
    for i in range(2):
        xm = _lin(xm, p["Wl"][i], p["bl"][i])
        x_first = _lin(x_first, p["Wf"][i], p["bf"][i])
        x_last = _lin(x_last, p["Wlst"][i], p["blst"][i])
        x_depot = _lin(x_depot, p["Wd"][i], p["bd"][i])
        x_vis = _lin(x_vis, p["Wvv"][i], p["bvv"][i])
        if i < 1:
            xm = _silu(xm); x_first = _silu(x_first); x_last = _silu(x_last)
            x_depot = _silu(x_depot); x_vis = _silu(x_vis)
        else:
            q = (x_vis[None, :, :] * visited[:, :, None]).sum(1) / visited.sum(1)[:, None]
            q = q + x_first[:, 0] + x_last[:, 0] + x_depot[:, 0]
            xm = (q @ xm.T).reshape(bsz, -1, 16)
            xm = jax.nn.softmax(xm, axis=-1)
    return (x_emb, xm)


# speedup vs baseline: 3.0537x; 3.0537x over previous
import functools

import jax
import jax.numpy as jnp
from jax import lax
from jax.experimental import pallas as pl
from jax.experimental.pallas import tpu as pltpu
from jax.experimental.pallas import tpu_sc as plsc

_N = 10000
_E = 160000
_U = 48
_DEPTH = 12
_B = 64
_KS = 16

_CHUNK = 80
_NCHUNK = _E // _CHUNK
_NW = 32
_NP = 10240
_RPS = _NP // 16


def _silu(t):
    return t / (1.0 + jnp.exp(-t))


@functools.cache
def _mesh():
    return plsc.VectorSubcoreMesh(
        core_axis_name="c", subcore_axis_name="s", num_cores=2, num_subcores=16)


@functools.cache
def _sc_edge_kernel():
    return pl.kernel(
        _sc_edge_body,
        out_type=(
            jax.ShapeDtypeStruct((_E, _U), jnp.float32),
            jax.ShapeDtypeStruct((2, _NP, 128), jnp.float32),
        ),
        mesh=_mesh(),
        scratch_types=[
            pltpu.VMEM((_CHUNK,), jnp.int32),
            pltpu.VMEM((_CHUNK,), jnp.int32),
            pltpu.VMEM((_CHUNK, 128), jnp.float32),
            pltpu.VMEM((_CHUNK, 128), jnp.float32),
            pltpu.VMEM((_CHUNK, _U), jnp.float32),
            pltpu.VMEM_SHARED((_NP, 128), jnp.float32),
            pltpu.SemaphoreType.DMA,
        ],
    )


def _sc_edge_body(t1, t2, w0, ei0, ei1, g_out, agg_out,
                  idx0_v, idx1_v, d1, d2, gbuf, accum, sem):
    c = lax.axis_index("c")
    s = lax.axis_index("s")
    wid = c * 16 + s
    zero16 = jnp.zeros((16,), jnp.float32)

    def _zrow(i, carry):
        for jj in range(8):
            d1[i, pl.ds(jj * 16, 16)] = zero16
        return carry

    lax.fori_loop(0, _CHUNK, _zrow, 0)
    base0 = s * _RPS
    for r in range(_RPS // _CHUNK):
        pltpu.sync_copy(d1, accum.at[pl.ds(base0 + r * _CHUNK, _CHUNK), :])
    plsc.subcore_barrier()

    def _chunk(t, carry):
        k = wid + t * _NW
        base = k * _CHUNK
        pltpu.sync_copy(ei0.at[pl.ds(base, _CHUNK)], idx0_v)
        pltpu.sync_copy(ei1.at[pl.ds(base, _CHUNK)], idx1_v)
        cps = [pltpu.async_copy(w0.at[pl.ds(base, _CHUNK), :], gbuf, sem),
               pltpu.async_copy(t1.at[idx1_v], d1, sem),
               pltpu.async_copy(t2.at[idx0_v], d2, sem)]
        for cp in cps:
            cp.wait()

        def _row(i, carry2):
            for jj in range(3):
                sl = pl.ds(jj * 16, 16)
                sh = pl.ds(48 + jj * 16, 16)
                wv = gbuf[i, sl]
                tv = d1[i, sl] / (1.0 + jnp.exp(-wv))
                gv = d2[i, sl] + d1[i, sh]
                d1[i, sl] = tv
                gbuf[i, sl] = gv
                d1[i, sh] = zero16
            return carry2

        lax.fori_loop(0, _CHUNK, _row, 0)
        pltpu.sync_copy(gbuf, g_out.at[pl.ds(base, _CHUNK), :])
        pltpu.sync_copy(d1, accum.at[idx0_v], add=True)
        return carry

    trips = (_NCHUNK - wid + _NW - 1) // _NW
    lax.fori_loop(0, trips, _chunk, 0)
    plsc.subcore_barrier()
    pltpu.sync_copy(accum.at[pl.ds(s * _RPS, _RPS), :],
                    agg_out.at[c, pl.ds(s * _RPS, _RPS), :])


@functools.cache
def _sc_counts_kernel():
    return pl.kernel(
        _sc_counts_body,
        out_type=jax.ShapeDtypeStruct((2, _NP, 128), jnp.float32),
        mesh=_mesh(),
        scratch_types=[
            pltpu.VMEM((_CHUNK,), jnp.int32),
            pltpu.VMEM((_CHUNK, 128), jnp.float32),
            pltpu.VMEM_SHARED((_NP, 128), jnp.float32),
        ],
    )


def _sc_counts_body(ei0, cnt_out, idx0_v, ones_v, accum):
    c = lax.axis_index("c")
    s = lax.axis_index("s")
    wid = c * 16 + s
    one16 = jnp.ones((16,), jnp.float32)
    zero16 = jnp.zeros((16,), jnp.float32)

    def _zrow(i, carry):
        for jj in range(8):
            ones_v[i, pl.ds(jj * 16, 16)] = zero16
        return carry

    lax.fori_loop(0, _CHUNK, _zrow, 0)
    base0 = s * _RPS
    for r in range(_RPS // _CHUNK):
        pltpu.sync_copy(ones_v, accum.at[pl.ds(base0 + r * _CHUNK, _CHUNK), :])

    def _frow(i, carry):
        for jj in range(8):
            ones_v[i, pl.ds(jj * 16, 16)] = one16
        return carry

    lax.fori_loop(0, _CHUNK, _frow, 0)
    plsc.subcore_barrier()

    def _chunk(t, carry):
        k = wid + t * _NW
        base = k * _CHUNK
        pltpu.sync_copy(ei0.at[pl.ds(base, _CHUNK)], idx0_v)
        pltpu.sync_copy(ones_v, accum.at[idx0_v], add=True)
        return carry

    trips = (_NCHUNK - wid + _NW - 1) // _NW
    lax.fori_loop(0, trips, _chunk, 0)
    plsc.subcore_barrier()
    pltpu.sync_copy(accum.at[pl.ds(s * _RPS, _RPS), :],
                    cnt_out.at[c, pl.ds(s * _RPS, _RPS), :])


def _pre_node_body(x_ref, wt_ref, b_ref, o_ref):
    def r(v):
        return v.astype(jnp.bfloat16).astype(jnp.float32)

    x = x_ref[...]
    wt = wt_ref[...]
    t = (r(x[:, 0:1]) * r(wt[0:1, :])
         + r(x[:, 1:2]) * r(wt[1:2, :])) + b_ref[...]
    o_ref[...] = _silu(t)


def _pre_edge_body(ea_ref, wt_ref, b_ref, o_ref):
    t = ea_ref[...] * wt_ref[...] + b_ref[...]
    o_ref[...] = _silu(t)


def _node_proj_body(h_ref, w1, w2, w3, w4, b1, b2, b3, b4, o1, ot1, ot2):
    h = h_ref[...]
    o1[...] = jnp.dot(h, w1[...], preferred_element_type=jnp.float32) + b1[...]
    x2 = jnp.dot(h, w2[...], preferred_element_type=jnp.float32) + b2[...]
    x3 = jnp.dot(h, w3[...], preferred_element_type=jnp.float32) + b3[...]
    x4 = jnp.dot(h, w4[...], preferred_element_type=jnp.float32) + b4[...]
    z32 = jnp.zeros((_N, 32), jnp.float32)
    z80 = jnp.zeros((_N, 80), jnp.float32)
    sl = pl.ds(0, _N)
    ot1[sl, :] = jnp.concatenate([x2, x4, z32], axis=1)
    ot2[sl, :] = jnp.concatenate([x3, z80], axis=1)


def _edge_stats_body(w0_ref, g_ref, wt_ref, be_ref, sum_ref, sq_ref):
    i = pl.program_id(0)
    t = (jnp.dot(w0_ref[...], wt_ref[...], preferred_element_type=jnp.float32)
         + be_ref[...] + g_ref[...])

    @pl.when(i == 0)
    def _():
        sum_ref[...] = jnp.zeros_like(sum_ref)
        sq_ref[...] = jnp.zeros_like(sq_ref)

    sum_ref[...] += jnp.sum(t, axis=0, keepdims=True)
    sq_ref[...] += jnp.sum(t * t, axis=0, keepdims=True)


def _edge_apply_body(w0_ref, g_ref, wt_ref, be_ref, sum_ref, sq_ref,
                     gam_ref, bet_ref, o_ref):
    w0 = w0_ref[...]
    t = (jnp.dot(w0, wt_ref[...], preferred_element_type=jnp.float32)
         + be_ref[...] + g_ref[...])
    mean = sum_ref[...] * (1.0 / _E)
    var = sq_ref[...] * (1.0 / _E) - mean * mean
    a = gam_ref[...] / jnp.sqrt(var + 1e-5)
    tn = (t - mean) * a + bet_ref[...]
    o_ref[...] = w0 + _silu(tn)


def _node_update_body(x0_ref, x1_ref, ap_ref, cnt_ref, gam_ref, bet_ref, o_ref):
    ap = ap_ref[...]
    agg = (ap[0, :_N, 0:_U] + ap[1, :_N, 0:_U]) / jnp.maximum(cnt_ref[...], 1.0)
    t = x1_ref[...] + agg
    m = jnp.mean(t, axis=0, keepdims=True)
    d = t - m
    v = jnp.mean(d * d, axis=0, keepdims=True)
    tn = d * (gam_ref[...] / jnp.sqrt(v + 1e-5)) + bet_ref[...]
    o_ref[...] = x0_ref[...] + _silu(tn)


def _q_body(xemb_ref, vis_ref, sol0_ref, sel0_ref,
            wf0, bf0, wf1, bf1, wl0, bl0, wl1, bl1,
            wd0, bd0, wd1, bd1, wv0, bv0, wv1, bv1, o_ref,
            r1_scr, r2_scr):
    def _gather(i, carry):
        i0 = sol0_ref[i]
        i1 = sel0_ref[i]
        r1_scr[pl.ds(i, 1), :] = xemb_ref[pl.ds(i0, 1), :]
        r2_scr[pl.ds(i, 1), :] = xemb_ref[pl.ds(i1, 1), :]
        return carry

    lax.fori_loop(0, _B, _gather, 0)
    xemb = xemb_ref[...]
    vis = vis_ref[...]
    v1 = _silu(jnp.dot(xemb, wv0[...], preferred_element_type=jnp.float32) + bv0[...])
    v2 = jnp.dot(v1, wv1[...], preferred_element_type=jnp.float32) + bv1[...]
    sv = jnp.dot(vis, v2, preferred_element_type=jnp.float32)
    den = jnp.sum(vis, axis=1, keepdims=True)
    qv = sv / den
    xf = r1_scr[...]
    xl = r2_scr[...]
    xd = xemb[0:1, :]
    xf2 = jnp.dot(_silu(jnp.dot(xf, wf0[...], preferred_element_type=jnp.float32) + bf0[...]),
                  wf1[...], preferred_element_type=jnp.float32) + bf1[...]
    xl2 = jnp.dot(_silu(jnp.dot(xl, wl0[...], preferred_element_type=jnp.float32) + bl0[...]),
                  wl1[...], preferred_element_type=jnp.float32) + bl1[...]
    xd2 = jnp.dot(_silu(jnp.dot(xd, wd0[...], preferred_element_type=jnp.float32) + bd0[...]),
                  wd1[...], preferred_element_type=jnp.float32) + bd1[...]
    o_ref[...] = qv + xf2 + xl2 + xd2


_FBLK = 3200
_FG = _FBLK // _KS


def _final_body(w_ref, qt_ref, wl0, bl0, wl1, bl1, o_ref):
    xm = jnp.dot(_silu(jnp.dot(w_ref[...], wl0[...], preferred_element_type=jnp.float32) + bl0[...]),
                 wl1[...], preferred_element_type=jnp.float32) + bl1[...]
    st = jnp.dot(xm, qt_ref[...], preferred_element_type=jnp.float32)
    s3 = st.reshape(_FG, _KS, _B)
    m = jnp.max(s3, axis=1, keepdims=True)
    p = jnp.exp(s3 - m)
    z = jnp.sum(p, axis=1, keepdims=True)
    o_ref[...] = p / z


def _full(a):
    return pl.BlockSpec(a, lambda i: tuple(0 for _ in a))


def kernel(x, edge_attr, visited, params, edge_index, solution, selected):
    p = params
    f32 = jnp.float32

    ei0 = edge_index[0]
    ei1 = edge_index[1]

    def b2(v):
        return v.reshape(1, _U)

    h = pl.pallas_call(
        _pre_node_body,
        out_shape=jax.ShapeDtypeStruct((_N, _U), f32),
    )(x, p["Wv0"].T, b2(p["bv0"]))

    epb = 20000
    w = pl.pallas_call(
        _pre_edge_body,
        grid=(_E // epb,),
        in_specs=[pl.BlockSpec((epb, 1), lambda i: (i, 0)),
                  _full((1, _U)), _full((1, _U))],
        out_specs=pl.BlockSpec((epb, _U), lambda i: (i, 0)),
        out_shape=jax.ShapeDtypeStruct((_E, _U), f32),
    )(edge_attr, p["We0"].T, b2(p["be0"]))

    cnt_parts = _sc_counts_kernel()(ei0)
    cnt = cnt_parts[0, :_N, 0:1] + cnt_parts[1, :_N, 0:1]

    node_proj = pl.pallas_call(
        _node_proj_body,
        out_shape=(jax.ShapeDtypeStruct((_N, _U), f32),
                   jax.ShapeDtypeStruct((_NP, 128), f32),
                   jax.ShapeDtypeStruct((_NP, 128), f32)),
    )

    eb = 16000
    edge_stats = pl.pallas_call(
        _edge_stats_body,
        grid=(_E // eb,),
        in_specs=[pl.BlockSpec((eb, _U), lambda i: (i, 0)),
                  pl.BlockSpec((eb, _U), lambda i: (i, 0)),
                  _full((_U, _U)), _full((1, _U))],
        out_specs=(_full((1, _U)), _full((1, _U))),
        out_shape=(jax.ShapeDtypeStruct((1, _U), f32),
                   jax.ShapeDtypeStruct((1, _U), f32)),
    )
    edge_apply = pl.pallas_call(
        _edge_apply_body,
        grid=(_E // eb,),
        in_specs=[pl.BlockSpec((eb, _U), lambda i: (i, 0)),
                  pl.BlockSpec((eb, _U), lambda i: (i, 0)),
                  _full((_U, _U)), _full((1, _U)), _full((1, _U)),
                  _full((1, _U)), _full((1, _U)), _full((1, _U))],
        out_specs=pl.BlockSpec((eb, _U), lambda i: (i, 0)),
        out_shape=jax.ShapeDtypeStruct((_E, _U), f32),
    )
    node_update = pl.pallas_call(
        _node_update_body,
        out_shape=jax.ShapeDtypeStruct((_N, _U), f32),
    )

    for i in range(_DEPTH):
        x1, t1, t2 = node_proj(
            h, p["Wv1"][i].T, p["Wv2"][i].T, p["Wv3"][i].T, p["Wv4"][i].T,
            b2(p["bv1"][i]), b2(p["bv2"][i]), b2(p["bv3"][i]), b2(p["bv4"][i]))
        g, agg = _sc_edge_kernel()(t1, t2, w, ei0, ei1)
        wet = p["We"][i].T
        bei = b2(p["be"][i])
        ssum, ssq = edge_stats(w, g, wet, bei)
        w = edge_apply(w, g, wet, bei, ssum, ssq,
                       b2(p["ebn_g"][i]), b2(p["ebn_b"][i]))
        h = node_update(h, x1, agg, cnt, b2(p["vbn_g"][i]), b2(p["vbn_b"][i]))

    xemb = h

    q = pl.pallas_call(
        _q_body,
        in_specs=[
            pl.BlockSpec(memory_space=pltpu.VMEM),
            pl.BlockSpec(memory_space=pltpu.VMEM),
            pl.BlockSpec(memory_space=pltpu.SMEM),
            pl.BlockSpec(memory_space=pltpu.SMEM),
        ] + [pl.BlockSpec(memory_space=pltpu.VMEM)] * 16,
        out_shape=jax.ShapeDtypeStruct((_B, _U), f32),
        scratch_shapes=[pltpu.VMEM((_B, _U), f32), pltpu.VMEM((_B, _U), f32)],
    )(xemb, visited, solution[:, 0], selected[:, 0],
      p["Wf"][0].T, b2(p["bf"][0]), p["Wf"][1].T, b2(p["bf"][1]),
      p["Wlst"][0].T, b2(p["blst"][0]), p["Wlst"][1].T, b2(p["blst"][1]),
      p["Wd"][0].T, b2(p["bd"][0]), p["Wd"][1].T, b2(p["bd"][1]),
      p["Wvv"][0].T, b2(p["bvv"][0]), p["Wvv"][1].T, b2(p["bvv"][1]))

    out9 = pl.pallas_call(
        _final_body,
        grid=(_E // _FBLK,),
        in_specs=[pl.BlockSpec((_FBLK, _U), lambda i: (i, 0)),
                  _full((_U, _B)),
                  _full((_U, _U)), _full((1, _U)),
                  _full((_U, _U)), _full((1, _U))],
        out_specs=pl.BlockSpec((_FG, _KS, _B), lambda i: (i, 0, 0)),
        out_shape=jax.ShapeDtypeStruct((_N, _KS, _B), f32),
    )(w, q.T, p["Wl"][0].T, b2(p["bl"][0]), p["Wl"][1].T, b2(p["bl"][1]))

    xm = jnp.transpose(out9, (2, 0, 1))
    return (xemb, xm)
